# baseline (speedup 1.0000x reference)
import functools

import jax
import jax.numpy as jnp
from jax import lax
from jax.experimental import pallas as pl
from jax.experimental.pallas import tpu as pltpu
from jax.experimental.pallas import tpu_sc as plsc

_DIM = 64
_LANES = 16
_NCORES = 2
_NSUB = 16
_NW = _NCORES * _NSUB
_BR = 4
_W = 128


def _make_sc_call(batch, nrows):
    bpw = batch // _NW
    ngrp = bpw // _LANES
    nsub = _LANES // _BR
    last_w0 = ((nrows - 64) >> 7) << 7
    tail0 = nrows - 64
    mesh = plsc.VectorSubcoreMesh(core_axis_name="c", subcore_axis_name="s")

    @functools.partial(
        pl.kernel,
        mesh=mesh,
        out_type=jax.ShapeDtypeStruct((_NW, _LANES), jnp.float32),
        compiler_params=pltpu.CompilerParams(needs_layout_passes=False),
        scratch_types=[
            pltpu.VMEM((bpw,), jnp.int32),
            pltpu.VMEM((_DIM, bpw), jnp.float32),
            pltpu.VMEM((_BR, 8, 8, _W), jnp.float32),
            pltpu.VMEM((_BR, 8, 8, _W), jnp.float32),
            pltpu.VMEM((64, _DIM), jnp.float32),
            pltpu.VMEM((_LANES,), jnp.float32),
            pltpu.SemaphoreType.DMA,
            pltpu.SemaphoreType.DMA,
            pltpu.SemaphoreType.DMA,
        ],
    )
    def sc_kernel(xt_hbm, y_hbm, ct_hbm, tail_hbm, out_hbm,
                  idx_v, x_v, c_v0, c_v1, tail_v, acc_v,
                  sem_x, sem_g0, sem_g1):
        wid = lax.axis_index("s") * _NCORES + lax.axis_index("c")
        base = wid * bpw
        slots = ((c_v0, sem_g0), (c_v1, sem_g1))

        pltpu.sync_copy(y_hbm.at[pl.ds(base, bpw)], idx_v)
        pltpu.sync_copy(tail_hbm, tail_v)
        cp_x = pltpu.async_copy(
            xt_hbm.at[pl.ds(0, _DIM), pl.ds(base, bpw)], x_v, sem_x)
        cp_x.wait()

        iota16 = lax.iota(jnp.int32, _LANES)
        kvecs = []
        for k in range(_DIM // _LANES):
            c_vec = iota16 + (k * _LANES)
            kvecs.append((c_vec >> 3, c_vec & 7, c_vec))

        def issue_sub(w0s, j, cref, sem):
            for q in range(_BR):
                w0 = pl.multiple_of(w0s[j * _BR + q], _W)
                for a in range(8):
                    pltpu.async_copy(
                        ct_hbm.at[a, pl.ds(0, 8), pl.ds(w0, _W)],
                        cref.at[q, a],
                        sem,
                    )

        def drain_sub(cref, sem):
            for q in range(_BR):
                pltpu.make_async_copy(
                    ct_hbm.at[pl.ds(0, 8), pl.ds(0, 8), pl.ds(0, _W)],
                    cref.at[q],
                    sem,
                ).wait()

        def compute_sub(col0, rv, lv, j, cref, accs_in):
            new = list(accs_in)
            for q in range(_BR):
                i = j * _BR + q
                r = rv[i]
                lq = lv[i]
                rt = jnp.clip(r - tail0, 0, 63)
                is_tail = r >= tail0
                for k in range(_DIM // _LANES):
                    a_vec, s_vec, c_vec = kvecs[k]
                    win = plsc.load_gather(
                        cref, [jnp.full((_LANES,), q, jnp.int32),
                               a_vec, s_vec,
                               jnp.full((_LANES,), lq, jnp.int32)])
                    tl = tail_v[rt, pl.ds(k * _LANES, _LANES)]
                    cv = jnp.where(is_tail, tl, win)
                    xv = plsc.load_gather(
                        x_v, [c_vec,
                              jnp.full((_LANES,), col0 + i, jnp.int32)])
                    d = xv - cv
                    new[k] = new[k] + d * d
            return tuple(new)

        zeros = jnp.zeros((_LANES,), jnp.float32)

        def body(g, accs_in):
            off = g * _LANES
            rv = idx_v[pl.ds(off, _LANES)]
            w0v = jnp.minimum((rv >> 7) << 7, last_w0)
            lv = jnp.minimum(rv - w0v, _W - 1)
            accs = accs_in
            for j in range(2):
                issue_sub(w0v, j, *slots[j])
            for j in range(nsub):
                cref, sem = slots[j % 2]
                drain_sub(cref, sem)
                accs = compute_sub(off, rv, lv, j, cref, accs)
                if j + 2 < nsub:
                    issue_sub(w0v, j + 2, cref, sem)
            return accs

        accs = lax.fori_loop(0, ngrp, body, (zeros, zeros, zeros, zeros))

        acc_v[...] = accs[0] + accs[1] + accs[2] + accs[3]
        pltpu.sync_copy(acc_v, out_hbm.at[wid])

    return sc_kernel


def kernel(x, y, centers):
    batch, dim = x.shape
    nrows = centers.shape[0]
    xt = x.T
    ct = centers.T.reshape(dim // 8, 8, nrows)
    tail = lax.slice(centers, (nrows - 64, 0), (nrows, dim))
    partials = _make_sc_call(batch, nrows)(
        xt, y.astype(jnp.int32), ct, tail)
    return jnp.sum(partials) / (batch * dim)


# speedup vs baseline: 1.0531x; 1.0531x over previous
import functools

import jax
import jax.numpy as jnp
from jax import lax
from jax.experimental import pallas as pl
from jax.experimental.pallas import tpu as pltpu
from jax.experimental.pallas import tpu_sc as plsc

_DIM = 64
_LANES = 16
_NCORES = 2
_NSUB = 16
_NW = _NCORES * _NSUB


def _make_sc_call(batch):
    bpw = batch // _NW
    nblk = bpw // _LANES
    mesh = plsc.VectorSubcoreMesh(core_axis_name="c", subcore_axis_name="s")

    slot_shape = (_LANES // 8, 8, _DIM)

    @functools.partial(
        pl.kernel,
        mesh=mesh,
        out_type=jax.ShapeDtypeStruct((_NW, _LANES), jnp.float32),
        scratch_types=[
            pltpu.VMEM((bpw,), jnp.int32),
            pltpu.VMEM((bpw, _DIM), jnp.float32),
            pltpu.VMEM(slot_shape, jnp.float32),
            pltpu.VMEM(slot_shape, jnp.float32),
            pltpu.VMEM((_LANES,), jnp.float32),
            pltpu.SemaphoreType.DMA,
            pltpu.SemaphoreType.DMA,
            pltpu.SemaphoreType.DMA,
        ],
    )
    def sc_kernel(x_hbm, y_hbm, centers_hbm, out_hbm, idx_v, x_v,
                  c_v0, c_v1, acc_v, sem_x, sem_g0, sem_g1):
        wid = lax.axis_index("s") * _NCORES + lax.axis_index("c")
        base = wid * bpw
        slots = ((c_v0, sem_g0), (c_v1, sem_g1))

        pltpu.sync_copy(y_hbm.at[pl.ds(base, bpw)], idx_v)
        cp_x = pltpu.async_copy(x_hbm.at[pl.ds(base, bpw)], x_v, sem_x)

        def issue_block(g, cref, sem):
            rv = idx_v[pl.ds(g * _LANES, _LANES)]
            tv = rv >> 3
            sv = rv & 7
            for i in range(_LANES):
                pltpu.async_copy(
                    centers_hbm.at[tv[i], sv[i]],
                    cref.at[i // 8, i % 8],
                    sem,
                )

        def drain_block(cref, sem):
            pltpu.make_async_copy(
                centers_hbm.at[pl.ds(0, _LANES // 8)], cref, sem).wait()

        def compute_block(g, cref, accs_in):
            off = g * _LANES
            new = list(accs_in)
            for i in range(_LANES):
                for k in range(_DIM // _LANES):
                    d = (x_v[off + i, pl.ds(k * _LANES, _LANES)]
                         - cref[i // 8, i % 8, pl.ds(k * _LANES, _LANES)])
                    new[k] = new[k] + d * d
            return tuple(new)

        issue_block(0, c_v0, sem_g0)
        issue_block(1, c_v1, sem_g1)
        cp_x.wait()

        zeros = jnp.zeros((_LANES,), jnp.float32)

        def body(it, accs_in):
            g = it * 2
            accs = accs_in
            for b, (cref, sem) in enumerate(slots):
                drain_block(cref, sem)
                accs = compute_block(g + b, cref, accs)

                @pl.when(g + b + 2 < nblk)
                def _():
                    issue_block(g + b + 2, cref, sem)

            return accs

        accs = lax.fori_loop(0, nblk // 2, body, (zeros, zeros, zeros, zeros))

        acc_v[...] = accs[0] + accs[1] + accs[2] + accs[3]
        pltpu.sync_copy(acc_v, out_hbm.at[wid])

    return sc_kernel


def kernel(x, y, centers):
    batch, dim = x.shape
    nrows = centers.shape[0]
    centers3 = centers.reshape(nrows // 8, 8, dim)
    partials = _make_sc_call(batch)(x, y.astype(jnp.int32), centers3)
    return jnp.sum(partials) / (batch * dim)


# speedup vs baseline: 1.0557x; 1.0024x over previous
import functools

import jax
import jax.numpy as jnp
from jax import lax
from jax.experimental import pallas as pl
from jax.experimental.pallas import tpu as pltpu
from jax.experimental.pallas import tpu_sc as plsc

_DIM = 64
_LANES = 16
_NCORES = 2
_NSUB = 16
_NW = _NCORES * _NSUB


_BLK = 32


def _make_sc_call(batch):
    bpw = batch // _NW
    nblk = bpw // _BLK
    mesh = plsc.VectorSubcoreMesh(core_axis_name="c", subcore_axis_name="s")

    slot_shape = (_BLK // 8, 8, _DIM)

    @functools.partial(
        pl.kernel,
        mesh=mesh,
        out_type=jax.ShapeDtypeStruct((_NW, _LANES), jnp.float32),
        scratch_types=[
            pltpu.VMEM((bpw,), jnp.int32),
            pltpu.VMEM((bpw, _DIM), jnp.float32),
            pltpu.VMEM(slot_shape, jnp.float32),
            pltpu.VMEM(slot_shape, jnp.float32),
            pltpu.VMEM((_LANES,), jnp.float32),
            pltpu.SemaphoreType.DMA,
            pltpu.SemaphoreType.DMA,
            pltpu.SemaphoreType.DMA,
        ],
    )
    def sc_kernel(x_hbm, y_hbm, centers_hbm, out_hbm, idx_v, x_v,
                  c_v0, c_v1, acc_v, sem_x, sem_g0, sem_g1):
        wid = lax.axis_index("s") * _NCORES + lax.axis_index("c")
        base = wid * bpw
        slots = ((c_v0, sem_g0), (c_v1, sem_g1))

        pltpu.sync_copy(y_hbm.at[pl.ds(base, bpw)], idx_v)
        cp_x = pltpu.async_copy(x_hbm.at[pl.ds(base, bpw)], x_v, sem_x)

        def issue_block(g, cref, sem):
            for i0 in range(0, _BLK, _LANES):
                rv = idx_v[pl.ds(g * _BLK + i0, _LANES)]
                tv = rv >> 3
                sv = rv & 7
                for i in range(_LANES):
                    pltpu.async_copy(
                        centers_hbm.at[tv[i], sv[i]],
                        cref.at[(i0 + i) // 8, (i0 + i) % 8],
                        sem,
                    )

        def drain_block(cref, sem):
            pltpu.make_async_copy(
                centers_hbm.at[pl.ds(0, _BLK // 8)], cref, sem).wait()

        def compute_block(g, cref, accs_in):
            off = g * _BLK
            new = list(accs_in)
            for i in range(_BLK):
                for k in range(_DIM // _LANES):
                    d = (x_v[off + i, pl.ds(k * _LANES, _LANES)]
                         - cref[i // 8, i % 8, pl.ds(k * _LANES, _LANES)])
                    new[k] = new[k] + d * d
            return tuple(new)

        issue_block(0, c_v0, sem_g0)
        issue_block(1, c_v1, sem_g1)
        cp_x.wait()

        zeros = jnp.zeros((_LANES,), jnp.float32)

        def body(it, accs_in):
            g = it * 2
            accs = accs_in
            for b, (cref, sem) in enumerate(slots):
                drain_block(cref, sem)
                accs = compute_block(g + b, cref, accs)

                @pl.when(g + b + 2 < nblk)
                def _():
                    issue_block(g + b + 2, cref, sem)

            return accs

        accs = lax.fori_loop(0, nblk // 2, body, (zeros, zeros, zeros, zeros))

        acc_v[...] = accs[0] + accs[1] + accs[2] + accs[3]
        pltpu.sync_copy(acc_v, out_hbm.at[wid])

    return sc_kernel


def kernel(x, y, centers):
    batch, dim = x.shape
    nrows = centers.shape[0]
    centers3 = centers.reshape(nrows // 8, 8, dim)
    partials = _make_sc_call(batch)(x, y.astype(jnp.int32), centers3)
    return jnp.sum(partials) / (batch * dim)
